# baseline (speedup 1.0000x reference)
import functools

import jax
import jax.numpy as jnp
from jax import lax
from jax.experimental import pallas as pl
from jax.experimental.pallas import tpu as pltpu
from jax.experimental.pallas import tpu_sc as plsc

N = 10000
E = 320000
D = 128
NUM_LAYERS = 4
EPS = 1e-5

NC = 2
NS = 16
NW = NC * NS
N_PAD = 10240
E_PAD = 327680
ROWS_PER_W = E_PAD // NW // 128
NROW_PER_TILE = N_PAD // NS

BLK = 256
GRID = N_PAD // BLK


def _sc_mesh():
    return plsc.VectorSubcoreMesh(core_axis_name="c", subcore_axis_name="s",
                                  num_cores=NC, num_subcores=NS)


def _deg_body(dst_hbm, degp_hbm, ones_v, idx_v, acc_sh):
    c = lax.axis_index("c")
    s = lax.axis_index("s")
    w = c * NS + s

    @pl.loop(0, 128)
    def _z(r):
        for kk in range(8):
            ones_v[r, pl.ds(kk * 16, 16)] = jnp.zeros((16,), jnp.float32)

    for t in range(NROW_PER_TILE // 128):
        pltpu.sync_copy(ones_v, acc_sh.at[pl.ds(s * NROW_PER_TILE + t * 128, 128), :])
    plsc.subcore_barrier()

    @pl.loop(0, 128)
    def _o(r):
        for kk in range(8):
            ones_v[r, pl.ds(kk * 16, 16)] = jnp.ones((16,), jnp.float32)

    pltpu.sync_copy(dst_hbm.at[pl.ds(w * ROWS_PER_W, ROWS_PER_W), :], idx_v)

    @pl.loop(0, ROWS_PER_W)
    def _scat(j):
        pltpu.sync_copy(ones_v, acc_sh.at[idx_v.at[j]], add=True)

    plsc.subcore_barrier()
    for t in range(NROW_PER_TILE // 128):
        r0 = s * NROW_PER_TILE + t * 128
        pltpu.sync_copy(acc_sh.at[pl.ds(r0, 128), :], degp_hbm.at[c, pl.ds(r0, 128), :])


def _deg_counts(dst2d):
    k = pl.kernel(
        _deg_body,
        out_type=jax.ShapeDtypeStruct((NC, N_PAD, D), jnp.float32),
        mesh=_sc_mesh(),
        scratch_types=[
            pltpu.VMEM((128, D), jnp.float32),
            pltpu.VMEM((ROWS_PER_W, 128), jnp.int32),
            pltpu.VMEM_SHARED((N_PAD, D), jnp.float32),
        ],
    )
    return k(dst2d)


def _agg_body(hh_hbm, src_hbm, dst_hbm, zeros_hbm, accp_hbm, rows_v, sidx_v,
              didx_v, acc_sh, sem0, sem1):
    c = lax.axis_index("c")
    s = lax.axis_index("s")
    w = c * NS + s

    pltpu.sync_copy(zeros_hbm.at[pl.ds(s * NROW_PER_TILE, NROW_PER_TILE), :],
                    acc_sh.at[pl.ds(s * NROW_PER_TILE, NROW_PER_TILE), :])
    plsc.subcore_barrier()

    HB = ROWS_PER_W // 2
    for half in range(2):
        base = w * ROWS_PER_W + half * HB
        pltpu.sync_copy(src_hbm.at[pl.ds(base, HB), :], sidx_v)
        pltpu.sync_copy(dst_hbm.at[pl.ds(base, HB), :], didx_v)
        pltpu.async_copy(hh_hbm.at[sidx_v.at[0]], rows_v.at[0], sem0)

        @pl.loop(0, HB // 2)
        def _edge(t):
            j = t * 2
            pltpu.async_copy(hh_hbm.at[sidx_v.at[j + 1]], rows_v.at[1], sem1)
            pltpu.make_async_copy(hh_hbm.at[sidx_v.at[j]], rows_v.at[0], sem0).wait()
            pltpu.sync_copy(rows_v.at[0], acc_sh.at[didx_v.at[j]], add=True)

            @pl.when(t < HB // 2 - 1)
            def _():
                pltpu.async_copy(hh_hbm.at[sidx_v.at[j + 2]], rows_v.at[0], sem0)

            pltpu.make_async_copy(hh_hbm.at[sidx_v.at[j + 1]], rows_v.at[1], sem1).wait()
            pltpu.sync_copy(rows_v.at[1], acc_sh.at[didx_v.at[j + 1]], add=True)

    plsc.subcore_barrier()
    for t in range(NROW_PER_TILE // 128):
        r0 = s * NROW_PER_TILE + t * 128
        pltpu.sync_copy(acc_sh.at[pl.ds(r0, 128), :], accp_hbm.at[c, pl.ds(r0, 128), :])


def _aggregate(hh, src2d, dst2d, zeros):
    k = pl.kernel(
        _agg_body,
        out_type=jax.ShapeDtypeStruct((NC, N_PAD, D), jnp.float32),
        mesh=_sc_mesh(),
        scratch_types=[
            pltpu.VMEM((2, 128, D), jnp.float32),
            pltpu.VMEM((ROWS_PER_W // 2, 128), jnp.int32),
            pltpu.VMEM((ROWS_PER_W // 2, 128), jnp.int32),
            pltpu.VMEM_SHARED((N_PAD, D), jnp.float32),
            pltpu.SemaphoreType.DMA,
            pltpu.SemaphoreType.DMA,
        ],
    )
    return k(hh, src2d, dst2d, zeros)


def _dinv_body(degp_ref, dinv_ref):
    deg = 1.0 + degp_ref[0, :, 0:1] + degp_ref[1, :, 0:1]
    dinv = lax.rsqrt(deg)
    row = lax.broadcasted_iota(jnp.int32, (N_PAD, 1), 0)
    dinv_ref[...] = jnp.where(row < N, dinv, 0.0)


def _dinv_kernel(degp):
    return pl.pallas_call(
        _dinv_body,
        out_shape=jax.ShapeDtypeStruct((N_PAD, 1), jnp.float32),
    )(degp)


def _bf16_mm(a, b):
    return lax.dot_general(a.astype(jnp.bfloat16), b.astype(jnp.bfloat16),
                           (((1,), (0,)), ((), ())),
                           preferred_element_type=jnp.float32)


def _mm1_body(z_ref, w_ref, dinv_ref, hh_ref):
    hh_ref[...] = _bf16_mm(z_ref[...], w_ref[...]) * dinv_ref[...]


def _mm1(z, W, dinv):
    return pl.pallas_call(
        _mm1_body,
        grid=(GRID,),
        in_specs=[
            pl.BlockSpec((BLK, D), lambda i: (i, 0)),
            pl.BlockSpec((D, D), lambda i: (0, 0)),
            pl.BlockSpec((BLK, 1), lambda i: (i, 0)),
        ],
        out_specs=pl.BlockSpec((BLK, D), lambda i: (i, 0)),
        out_shape=jax.ShapeDtypeStruct((N_PAD, D), jnp.float32),
    )(z, W, dinv)


def _mm_bn_body(r_ref, stats_ref, g_ref, be_ref, w_ref, dinv_ref, hh_ref):
    mu = stats_ref[0:1, :] * (1.0 / N)
    var = stats_ref[1:2, :] * (1.0 / N) - mu * mu
    sinv = lax.rsqrt(var + EPS)
    zbn = (r_ref[...] - mu) * sinv * g_ref[...] + be_ref[...]
    hh_ref[...] = _bf16_mm(zbn, w_ref[...]) * dinv_ref[...]


def _mm_bn(r, stats, g, be, W, dinv):
    return pl.pallas_call(
        _mm_bn_body,
        grid=(GRID,),
        in_specs=[
            pl.BlockSpec((BLK, D), lambda i: (i, 0)),
            pl.BlockSpec((8, D), lambda i: (0, 0)),
            pl.BlockSpec((1, D), lambda i: (0, 0)),
            pl.BlockSpec((1, D), lambda i: (0, 0)),
            pl.BlockSpec((D, D), lambda i: (0, 0)),
            pl.BlockSpec((BLK, 1), lambda i: (i, 0)),
        ],
        out_specs=pl.BlockSpec((BLK, D), lambda i: (i, 0)),
        out_shape=jax.ShapeDtypeStruct((N_PAD, D), jnp.float32),
    )(r, stats, g, be, W, dinv)


def _post_body(accp_ref, hh_ref, dinv_ref, b_ref, r_ref, stats_ref):
    i = pl.program_id(0)
    a = accp_ref[0] + accp_ref[1]
    o = dinv_ref[...] * (a + hh_ref[...]) + b_ref[...]
    r = jnp.maximum(o, 0.0)
    row = lax.broadcasted_iota(jnp.int32, (BLK, D), 0) + i * BLK
    rm = jnp.where(row < N, r, 0.0)
    r_ref[...] = rm

    @pl.when(i == 0)
    def _():
        stats_ref[...] = jnp.zeros((8, D), jnp.float32)

    stats_ref[0:1, :] += jnp.sum(rm, axis=0, keepdims=True)
    stats_ref[1:2, :] += jnp.sum(rm * rm, axis=0, keepdims=True)


def _post(accp, hh, dinv, b):
    return pl.pallas_call(
        _post_body,
        grid=(GRID,),
        in_specs=[
            pl.BlockSpec((NC, BLK, D), lambda i: (0, i, 0)),
            pl.BlockSpec((BLK, D), lambda i: (i, 0)),
            pl.BlockSpec((BLK, 1), lambda i: (i, 0)),
            pl.BlockSpec((1, D), lambda i: (0, 0)),
        ],
        out_specs=[
            pl.BlockSpec((BLK, D), lambda i: (i, 0)),
            pl.BlockSpec((8, D), lambda i: (0, 0)),
        ],
        out_shape=[
            jax.ShapeDtypeStruct((N_PAD, D), jnp.float32),
            jax.ShapeDtypeStruct((8, D), jnp.float32),
        ],
        compiler_params=pltpu.CompilerParams(
            dimension_semantics=("arbitrary",)),
    )(accp, hh, dinv, b)


def _bn_apply_body(r_ref, stats_ref, g_ref, be_ref, z_ref):
    mu = stats_ref[0:1, :] * (1.0 / N)
    var = stats_ref[1:2, :] * (1.0 / N) - mu * mu
    sinv = lax.rsqrt(var + EPS)
    z_ref[...] = (r_ref[...] - mu) * sinv * g_ref[...] + be_ref[...]


def _bn_apply(r, stats, g, be):
    return pl.pallas_call(
        _bn_apply_body,
        grid=(GRID,),
        in_specs=[
            pl.BlockSpec((BLK, D), lambda i: (i, 0)),
            pl.BlockSpec((8, D), lambda i: (0, 0)),
            pl.BlockSpec((1, D), lambda i: (0, 0)),
            pl.BlockSpec((1, D), lambda i: (0, 0)),
        ],
        out_specs=pl.BlockSpec((BLK, D), lambda i: (i, 0)),
        out_shape=jax.ShapeDtypeStruct((N_PAD, D), jnp.float32),
    )(r, stats, g, be)


def kernel(x, edge_index, W1, b1, g1, be1, W2, b2, g2, be2, W3, b3, g3, be3,
           W4, b4, g4, be4):
    src = edge_index[0]
    dst = edge_index[1]
    npad_e = E_PAD - E
    spread = N + (jnp.arange(npad_e, dtype=jnp.int32) % (N_PAD - N))
    src2d = jnp.concatenate([src, spread]).reshape(E_PAD // 128, 128)
    dst2d = jnp.concatenate([dst, spread]).reshape(E_PAD // 128, 128)

    xp = jnp.concatenate([x, jnp.zeros((N_PAD - N, D), x.dtype)], axis=0)

    degp = _deg_counts(dst2d)
    dinv = _dinv_kernel(degp)

    params = [(W1, b1, g1, be1), (W2, b2, g2, be2),
              (W3, b3, g3, be3), (W4, b4, g4, be4)]

    zeros = jnp.zeros((N_PAD, D), jnp.float32)

    r = None
    stats = None
    for li, (W, b, g, be) in enumerate(params):
        if li == 0:
            hh = _mm1(xp, W, dinv)
        else:
            hh = _mm_bn(r, stats, params[li - 1][2].reshape(1, D),
                        params[li - 1][3].reshape(1, D), W, dinv)
        accp = _aggregate(hh, src2d, dst2d, zeros)
        r, stats = _post(accp, hh, dinv, b.reshape(1, D))

    z = _bn_apply(r, stats, g4.reshape(1, D), be4.reshape(1, D))
    return z[:N]


# speedup vs baseline: 1.0163x; 1.0163x over previous
import functools

import jax
import jax.numpy as jnp
from jax import lax
from jax.experimental import pallas as pl
from jax.experimental.pallas import tpu as pltpu
from jax.experimental.pallas import tpu_sc as plsc

N = 10000
E = 320000
D = 128
NUM_LAYERS = 4
EPS = 1e-5

NC = 2
NS = 16
NW = NC * NS
N_PAD = 10240
E_PAD = 327680
ROWS_PER_W = E_PAD // NW // 128
NROW_PER_TILE = N_PAD // NS

BLK = 256
GRID = N_PAD // BLK


def _sc_mesh():
    return plsc.VectorSubcoreMesh(core_axis_name="c", subcore_axis_name="s",
                                  num_cores=NC, num_subcores=NS)


def _deg_body(dst_hbm, degp_hbm, ones_v, idx_v, acc_sh):
    c = lax.axis_index("c")
    s = lax.axis_index("s")
    w = c * NS + s

    @pl.loop(0, 128)
    def _z(r):
        for kk in range(8):
            ones_v[r, pl.ds(kk * 16, 16)] = jnp.zeros((16,), jnp.float32)

    for t in range(NROW_PER_TILE // 128):
        pltpu.sync_copy(ones_v, acc_sh.at[pl.ds(s * NROW_PER_TILE + t * 128, 128), :])
    plsc.subcore_barrier()

    @pl.loop(0, 128)
    def _o(r):
        for kk in range(8):
            ones_v[r, pl.ds(kk * 16, 16)] = jnp.ones((16,), jnp.float32)

    pltpu.sync_copy(dst_hbm.at[pl.ds(w * ROWS_PER_W, ROWS_PER_W), :], idx_v)

    @pl.loop(0, ROWS_PER_W)
    def _scat(j):
        pltpu.sync_copy(ones_v, acc_sh.at[idx_v.at[j]], add=True)

    plsc.subcore_barrier()
    for t in range(NROW_PER_TILE // 128):
        r0 = s * NROW_PER_TILE + t * 128
        pltpu.sync_copy(acc_sh.at[pl.ds(r0, 128), :], degp_hbm.at[c, pl.ds(r0, 128), :])


def _deg_counts(dst2d):
    k = pl.kernel(
        _deg_body,
        out_type=jax.ShapeDtypeStruct((NC, N_PAD, D), jnp.float32),
        mesh=_sc_mesh(),
        scratch_types=[
            pltpu.VMEM((128, D), jnp.float32),
            pltpu.VMEM((ROWS_PER_W, 128), jnp.int32),
            pltpu.VMEM_SHARED((N_PAD, D), jnp.float32),
        ],
    )
    return k(dst2d)


def _agg_body(hh_hbm, src_hbm, dst_hbm, accp_hbm, rows_v, sidx_v,
              didx_v, acc_sh, sem0, sem1):
    c = lax.axis_index("c")
    s = lax.axis_index("s")
    w = c * NS + s

    @pl.loop(0, 128)
    def _z(r):
        for kk in range(8):
            rows_v[0, r, pl.ds(kk * 16, 16)] = jnp.zeros((16,), jnp.float32)

    for t in range(NROW_PER_TILE // 128):
        pltpu.sync_copy(rows_v.at[0],
                        acc_sh.at[pl.ds(s * NROW_PER_TILE + t * 128, 128), :])
    plsc.subcore_barrier()

    HB = ROWS_PER_W // 2
    for half in range(2):
        base = w * ROWS_PER_W + half * HB
        pltpu.sync_copy(src_hbm.at[pl.ds(base, HB), :], sidx_v)
        pltpu.sync_copy(dst_hbm.at[pl.ds(base, HB), :], didx_v)
        pltpu.async_copy(hh_hbm.at[sidx_v.at[0]], rows_v.at[0], sem0)

        @pl.loop(0, HB // 2)
        def _edge(t):
            j = t * 2
            pltpu.async_copy(hh_hbm.at[sidx_v.at[j + 1]], rows_v.at[1], sem1)
            pltpu.make_async_copy(hh_hbm.at[sidx_v.at[j]], rows_v.at[0], sem0).wait()
            pltpu.sync_copy(rows_v.at[0], acc_sh.at[didx_v.at[j]], add=True)

            @pl.when(t < HB // 2 - 1)
            def _():
                pltpu.async_copy(hh_hbm.at[sidx_v.at[j + 2]], rows_v.at[0], sem0)

            pltpu.make_async_copy(hh_hbm.at[sidx_v.at[j + 1]], rows_v.at[1], sem1).wait()
            pltpu.sync_copy(rows_v.at[1], acc_sh.at[didx_v.at[j + 1]], add=True)

    plsc.subcore_barrier()
    for t in range(NROW_PER_TILE // 128):
        r0 = s * NROW_PER_TILE + t * 128
        pltpu.sync_copy(acc_sh.at[pl.ds(r0, 128), :], accp_hbm.at[c, pl.ds(r0, 128), :])


def _aggregate(hh, src2d, dst2d):
    k = pl.kernel(
        _agg_body,
        out_type=jax.ShapeDtypeStruct((NC, N_PAD, D), jnp.float32),
        mesh=_sc_mesh(),
        scratch_types=[
            pltpu.VMEM((2, 128, D), jnp.float32),
            pltpu.VMEM((ROWS_PER_W // 2, 128), jnp.int32),
            pltpu.VMEM((ROWS_PER_W // 2, 128), jnp.int32),
            pltpu.VMEM_SHARED((N_PAD, D), jnp.float32),
            pltpu.SemaphoreType.DMA,
            pltpu.SemaphoreType.DMA,
        ],
    )
    return k(hh, src2d, dst2d)


def _dinv_body(degp_ref, dinv_ref):
    deg = 1.0 + degp_ref[0, :, 0:1] + degp_ref[1, :, 0:1]
    dinv = lax.rsqrt(deg)
    row = lax.broadcasted_iota(jnp.int32, (N_PAD, 1), 0)
    dinv_ref[...] = jnp.where(row < N, dinv, 0.0)


def _dinv_kernel(degp):
    return pl.pallas_call(
        _dinv_body,
        out_shape=jax.ShapeDtypeStruct((N_PAD, 1), jnp.float32),
    )(degp)


def _bf16_mm(a, b):
    return lax.dot_general(a.astype(jnp.bfloat16), b.astype(jnp.bfloat16),
                           (((1,), (0,)), ((), ())),
                           preferred_element_type=jnp.float32)


def _mm1_body(z_ref, w_ref, dinv_ref, hh_ref):
    hh_ref[...] = _bf16_mm(z_ref[...], w_ref[...]) * dinv_ref[...]


def _mm1(z, W, dinv):
    return pl.pallas_call(
        _mm1_body,
        grid=(GRID,),
        in_specs=[
            pl.BlockSpec((BLK, D), lambda i: (i, 0)),
            pl.BlockSpec((D, D), lambda i: (0, 0)),
            pl.BlockSpec((BLK, 1), lambda i: (i, 0)),
        ],
        out_specs=pl.BlockSpec((BLK, D), lambda i: (i, 0)),
        out_shape=jax.ShapeDtypeStruct((N_PAD, D), jnp.float32),
    )(z, W, dinv)


def _mm_bn_body(r_ref, stats_ref, g_ref, be_ref, w_ref, dinv_ref, hh_ref):
    mu = stats_ref[0:1, :] * (1.0 / N)
    var = stats_ref[1:2, :] * (1.0 / N) - mu * mu
    sinv = lax.rsqrt(var + EPS)
    zbn = (r_ref[...] - mu) * sinv * g_ref[...] + be_ref[...]
    hh_ref[...] = _bf16_mm(zbn, w_ref[...]) * dinv_ref[...]


def _mm_bn(r, stats, g, be, W, dinv):
    return pl.pallas_call(
        _mm_bn_body,
        grid=(GRID,),
        in_specs=[
            pl.BlockSpec((BLK, D), lambda i: (i, 0)),
            pl.BlockSpec((8, D), lambda i: (0, 0)),
            pl.BlockSpec((1, D), lambda i: (0, 0)),
            pl.BlockSpec((1, D), lambda i: (0, 0)),
            pl.BlockSpec((D, D), lambda i: (0, 0)),
            pl.BlockSpec((BLK, 1), lambda i: (i, 0)),
        ],
        out_specs=pl.BlockSpec((BLK, D), lambda i: (i, 0)),
        out_shape=jax.ShapeDtypeStruct((N_PAD, D), jnp.float32),
    )(r, stats, g, be, W, dinv)


def _post_body(accp_ref, hh_ref, dinv_ref, b_ref, r_ref, stats_ref):
    i = pl.program_id(0)
    a = accp_ref[0] + accp_ref[1]
    o = dinv_ref[...] * (a + hh_ref[...]) + b_ref[...]
    r = jnp.maximum(o, 0.0)
    row = lax.broadcasted_iota(jnp.int32, (BLK, D), 0) + i * BLK
    rm = jnp.where(row < N, r, 0.0)
    r_ref[...] = rm

    @pl.when(i == 0)
    def _():
        stats_ref[...] = jnp.zeros((8, D), jnp.float32)

    stats_ref[0:1, :] += jnp.sum(rm, axis=0, keepdims=True)
    stats_ref[1:2, :] += jnp.sum(rm * rm, axis=0, keepdims=True)


def _post(accp, hh, dinv, b):
    return pl.pallas_call(
        _post_body,
        grid=(GRID,),
        in_specs=[
            pl.BlockSpec((NC, BLK, D), lambda i: (0, i, 0)),
            pl.BlockSpec((BLK, D), lambda i: (i, 0)),
            pl.BlockSpec((BLK, 1), lambda i: (i, 0)),
            pl.BlockSpec((1, D), lambda i: (0, 0)),
        ],
        out_specs=[
            pl.BlockSpec((BLK, D), lambda i: (i, 0)),
            pl.BlockSpec((8, D), lambda i: (0, 0)),
        ],
        out_shape=[
            jax.ShapeDtypeStruct((N_PAD, D), jnp.float32),
            jax.ShapeDtypeStruct((8, D), jnp.float32),
        ],
        compiler_params=pltpu.CompilerParams(
            dimension_semantics=("arbitrary",)),
    )(accp, hh, dinv, b)


def _bn_apply_body(r_ref, stats_ref, g_ref, be_ref, z_ref):
    mu = stats_ref[0:1, :] * (1.0 / N)
    var = stats_ref[1:2, :] * (1.0 / N) - mu * mu
    sinv = lax.rsqrt(var + EPS)
    z_ref[...] = (r_ref[...] - mu) * sinv * g_ref[...] + be_ref[...]


def _bn_apply(r, stats, g, be):
    return pl.pallas_call(
        _bn_apply_body,
        grid=(GRID,),
        in_specs=[
            pl.BlockSpec((BLK, D), lambda i: (i, 0)),
            pl.BlockSpec((8, D), lambda i: (0, 0)),
            pl.BlockSpec((1, D), lambda i: (0, 0)),
            pl.BlockSpec((1, D), lambda i: (0, 0)),
        ],
        out_specs=pl.BlockSpec((BLK, D), lambda i: (i, 0)),
        out_shape=jax.ShapeDtypeStruct((N_PAD, D), jnp.float32),
    )(r, stats, g, be)


def kernel(x, edge_index, W1, b1, g1, be1, W2, b2, g2, be2, W3, b3, g3, be3,
           W4, b4, g4, be4):
    src = edge_index[0]
    dst = edge_index[1]
    npad_e = E_PAD - E
    spread = N + (jnp.arange(npad_e, dtype=jnp.int32) % (N_PAD - N))
    src2d = jnp.concatenate([src, spread]).reshape(E_PAD // 128, 128)
    dst2d = jnp.concatenate([dst, spread]).reshape(E_PAD // 128, 128)

    xp = jnp.concatenate([x, jnp.zeros((N_PAD - N, D), x.dtype)], axis=0)

    degp = _deg_counts(dst2d)
    dinv = _dinv_kernel(degp)

    params = [(W1, b1, g1, be1), (W2, b2, g2, be2),
              (W3, b3, g3, be3), (W4, b4, g4, be4)]

    r = None
    stats = None
    for li, (W, b, g, be) in enumerate(params):
        if li == 0:
            hh = _mm1(xp, W, dinv)
        else:
            hh = _mm_bn(r, stats, params[li - 1][2].reshape(1, D),
                        params[li - 1][3].reshape(1, D), W, dinv)
        accp = _aggregate(hh, src2d, dst2d)
        r, stats = _post(accp, hh, dinv, b.reshape(1, D))

    z = _bn_apply(r, stats, g4.reshape(1, D), be4.reshape(1, D))
    return z[:N]
